# Initial kernel scaffold; baseline (speedup 1.0000x reference)
import jax
import jax.numpy as jnp
from jax.experimental import pallas as pl


def kernel(x, edge_index, nrm_g, nrm_b, gW1, gb1, ln_g, ln_b, in_proj, conv_w, conv_b, x_proj, dt_proj_W, dt_bias, A_log, Dp, rW1, rb1, rW2, rb2, xW1, xb1, xW2, xb2, out_proj):
    raise NotImplementedError("write your pallas kernel here")


# speedup vs baseline: 6.8680x; 6.8680x over previous
import functools

import jax
import jax.numpy as jnp
from jax import lax
from jax.experimental import pallas as pl
from jax.experimental.pallas import tpu as pltpu
from jax.experimental.pallas import tpu_sc as plsc

N = 2048
FIN = 3
G = 64
GF = G * FIN
GFP = 256
E = 4096
EA = E + N
D_MODEL = 128
D_STATE = 32
D_CONV = 4
D_INNER = 256
N_LAYERS = 3
DT_RANK = 8
BZ, SL = 4, 16
NC, NS = 2, 16
EPS = EA // (NC * NS)
ECH = 512
NECH = EA // ECH


def _prep_kernel(xrow_ref, dst_ref, m_ref, gcol_ref, bcol_ref, xs_ref):
    xr = xrow_ref[...]
    mm = m_ref[...]
    mean = jnp.dot(xr, mm, preferred_element_type=jnp.float32,
        precision=lax.Precision.HIGHEST)
    d = xr - mean
    var = jnp.dot(d * d, mm, preferred_element_type=jnp.float32,
        precision=lax.Precision.HIGHEST)
    xln = d * 1.0 / jnp.sqrt(var + 1e-5) * gcol_ref[...] + bcol_ref[...]
    dstv = dst_ref[...]
    chunks = []
    for k in range(N // 128):
        nid = lax.broadcasted_iota(jnp.int32, (128, 1), 0) + (k * 128)
        cnt = jnp.sum((dstv == nid).astype(jnp.float32), axis=1, keepdims=True)
        chunks.append(cnt)
    deg = jnp.concatenate(chunks, axis=0) + 1.0
    dinv = 1.0 / jnp.sqrt(deg)
    xs_ref[...] = jnp.concatenate(
        [xln * dinv, jnp.zeros((N, GFP - GF), jnp.float32)], axis=1)


def _sc_gather(xs, srcp):
    mesh = plsc.VectorSubcoreMesh(core_axis_name="c", subcore_axis_name="s")

    @functools.partial(
        pl.kernel,
        out_type=jax.ShapeDtypeStruct((EA, GFP), jnp.float32),
        mesh=mesh,
        scratch_types=[
            pltpu.VMEM((EPS,), jnp.int32),
            pltpu.VMEM((EPS, GFP), jnp.float32),
            pltpu.SemaphoreType.DMA,
        ],
    )
    def gat(xs_hbm, src_hbm, out_hbm, srcv, rows, sem):
        c = lax.axis_index("c")
        s = lax.axis_index("s")
        base = (c * NS + s) * EPS
        pltpu.sync_copy(src_hbm.at[pl.ds(base, EPS)], srcv)
        h1 = pltpu.async_copy(
            xs_hbm.at[srcv.at[pl.ds(0, 128)]], rows.at[pl.ds(0, 128)], sem)
        h2 = pltpu.async_copy(
            xs_hbm.at[srcv.at[pl.ds(128, EPS - 128)]],
            rows.at[pl.ds(128, EPS - 128)], sem)
        h1.wait()
        h2.wait()
        pltpu.sync_copy(rows, out_hbm.at[pl.ds(base, EPS)])

    return gat(xs, srcp)


def _agg_mm_kernel(msgs_ref, dstp_ref, ut_ref, acc_ref, deg_ref):
    j = pl.program_id(0)

    @pl.when(j == 0)
    def _init():
        acc_ref[...] = jnp.zeros_like(acc_ref)
        deg_ref[...] = jnp.zeros_like(deg_ref)

    dstc = dstp_ref[0]
    onehot = (dstc == lax.broadcasted_iota(jnp.int32, (ECH, N), 1)
              ).astype(jnp.float32)
    acc_ref[...] += lax.dot_general(
        msgs_ref[...], onehot, (((0,), (0,)), ((), ())),
        preferred_element_type=jnp.float32,
        precision=lax.Precision.HIGHEST)
    deg_ref[...] += jnp.sum(onehot, axis=0, keepdims=True)

    @pl.when(j == NECH - 1)
    def _fin():
        ut_ref[...] = acc_ref[...] * 1.0 / jnp.sqrt(deg_ref[...])


def _gcn_mm_kernel(u3_ref, gwt_ref, gb_ref, emb_ref):
    xg = u3_ref[0]
    ht = jnp.dot(gwt_ref[...], xg, preferred_element_type=jnp.float32,
        precision=lax.Precision.HIGHEST)
    ht = jnp.maximum(ht + gb_ref[...], 0.0)
    emb_ref[0] = jnp.sum(ht, axis=1, keepdims=True) * (1.0 / N)


def _silu(v):
    return v * jax.nn.sigmoid(v)


def _softplus(v):
    return jnp.maximum(v, 0.0) + jnp.log(1.0 + jnp.exp(-jnp.abs(v)))


def _mamba_kernel(emb_ref, lng_ref, lnb_ref, inproj_ref, convw_ref, convb_ref,
                  xpdt_ref, xpb_ref, xpc_ref, dtw_ref, dtb_ref, alogt_ref,
                  dp_ref, outproj_ref, rw1_ref, rb1_ref, rw2_ref, rb2_ref,
                  xw1_ref, xb1_ref, xw2_ref, xb2_ref, out_ref):
    t = emb_ref[...]
    for l in range(N_LAYERS):
        mean = jnp.mean(t, axis=1, keepdims=True)
        d = t - mean
        var = jnp.mean(d * d, axis=1, keepdims=True)
        tln = d * 1.0 / jnp.sqrt(var + 1e-5) * lng_ref[l][None, :] + lnb_ref[l][None, :]
        xz = jnp.dot(tln, inproj_ref[l], preferred_element_type=jnp.float32,
        precision=lax.Precision.HIGHEST)
        xi, z = xz[:, :D_INNER], xz[:, D_INNER:]
        xi3 = xi.reshape(BZ, SL, D_INNER)
        conv = xi3 * convw_ref[l, :, D_CONV - 1][None, None, :]
        for k in range(D_CONV - 1):
            sh = D_CONV - 1 - k
            shifted = jnp.concatenate(
                [jnp.zeros((BZ, sh, D_INNER), jnp.float32), xi3[:, :SL - sh, :]],
                axis=1)
            conv = conv + shifted * convw_ref[l, :, k][None, None, :]
        xi2 = _silu(conv.reshape(G, D_INNER) + convb_ref[l][None, :])
        dtr = jnp.dot(xi2, xpdt_ref[l], preferred_element_type=jnp.float32,
        precision=lax.Precision.HIGHEST)
        dt = _softplus(jnp.dot(dtr, dtw_ref[l], preferred_element_type=jnp.float32,
        precision=lax.Precision.HIGHEST)
                       + dtb_ref[l][None, :])
        bc = jnp.dot(xi2, xpb_ref[l], preferred_element_type=jnp.float32,
        precision=lax.Precision.HIGHEST)
        cc = jnp.dot(xi2, xpc_ref[l], preferred_element_type=jnp.float32,
        precision=lax.Precision.HIGHEST)
        at = -jnp.exp(alogt_ref[l])
        dt3 = dt.reshape(BZ, SL, D_INNER)
        u3 = xi2.reshape(BZ, SL, D_INNER)
        bc3 = bc.reshape(BZ, SL, D_STATE)
        cc3 = cc.reshape(BZ, SL, D_STATE)
        h = jnp.zeros((BZ, D_STATE, D_INNER), jnp.float32)
        ys = []
        for s_ in range(SL):
            dts = dt3[:, s_, :][:, None, :]
            da = jnp.exp(dts * at[None])
            us = u3[:, s_, :][:, None, :]
            bs = bc3[:, s_, :][:, :, None]
            h = da * h + dts * bs * us
            cs = cc3[:, s_, :][:, :, None]
            ys.append(jnp.sum(h * cs, axis=1))
        y = jnp.stack(ys, axis=1).reshape(G, D_INNER)
        y = y + xi2 * dp_ref[l][None, :]
        y = y * _silu(z)
        t = t + jnp.dot(y, outproj_ref[l], preferred_element_type=jnp.float32,
        precision=lax.Precision.HIGHEST)
    final = t.reshape(BZ, SL, D_MODEL)[:, SL - 1, :]
    f1 = jnp.maximum(jnp.dot(final, rw1_ref[...],
                             preferred_element_type=jnp.float32,
        precision=lax.Precision.HIGHEST)
                     + rb1_ref[...], 0.0)
    r = jnp.sum(f1 * rw2_ref[...], axis=1, keepdims=True) + rb2_ref[...]
    f2 = jnp.maximum(jnp.dot(final, xw1_ref[...],
                             preferred_element_type=jnp.float32,
        precision=lax.Precision.HIGHEST)
                     + xb1_ref[...], 0.0)
    xp = jnp.sum(f2 * xw2_ref[...], axis=1, keepdims=True) + xb2_ref[...]
    out_ref[...] = jnp.concatenate(
        [r, xp, jnp.zeros((BZ, D_MODEL - 2), jnp.float32)], axis=1)


def kernel(x, edge_index, nrm_g, nrm_b, gW1, gb1, ln_g, ln_b, in_proj, conv_w,
           conv_b, x_proj, dt_proj_W, dt_bias, A_log, Dp, rW1, rb1, rW2, rb2,
           xW1, xb1, xW2, xb2, out_proj):
    f32 = jnp.float32
    xrow = x.reshape(G, N, FIN).transpose(1, 0, 2).reshape(N, GF)
    src = edge_index[0]
    dst = edge_index[1]
    dst_row = dst.reshape(1, E)
    nid = jnp.arange(N, dtype=jnp.int32)
    srcp = jnp.concatenate([src, nid])
    dstp = jnp.concatenate([dst, nid]).reshape(NECH, ECH, 1)
    eye = jnp.eye(G, dtype=f32)
    mmat = jnp.repeat(jnp.repeat(eye, FIN, axis=0), FIN, axis=1) * (1.0 / FIN)
    gcol = jnp.tile(nrm_g, G).reshape(1, GF)
    bcol = jnp.tile(nrm_b, G).reshape(1, GF)
    gwt = gW1.T
    gbcol = gb1.reshape(D_MODEL, 1)
    alogt = A_log.transpose(0, 2, 1)
    xp_dt = x_proj[:, :, :DT_RANK]
    xp_b = x_proj[:, :, DT_RANK:DT_RANK + D_STATE]
    xp_c = x_proj[:, :, DT_RANK + D_STATE:]
    rb1r = rb1.reshape(1, D_MODEL // 2)
    rw2r = rW2.reshape(1, D_MODEL // 2)
    rb2r = rb2.reshape(1, 1)
    xb1r = xb1.reshape(1, D_MODEL // 2)
    xw2r = xW2.reshape(1, D_MODEL // 2)
    xb2r = xb2.reshape(1, 1)

    xs = pl.pallas_call(
        _prep_kernel,
        out_shape=jax.ShapeDtypeStruct((N, GFP), f32),
    )(xrow, dst_row, mmat, gcol, bcol)

    msgs = _sc_gather(xs, srcp)

    ut = pl.pallas_call(
        _agg_mm_kernel,
        grid=(NECH,),
        in_specs=[
            pl.BlockSpec((ECH, GFP), lambda j: (j, 0)),
            pl.BlockSpec((1, ECH, 1), lambda j: (j, 0, 0)),
        ],
        out_specs=pl.BlockSpec((GFP, N), lambda j: (0, 0)),
        out_shape=jax.ShapeDtypeStruct((GFP, N), f32),
        scratch_shapes=[
            pltpu.VMEM((GFP, N), f32),
            pltpu.VMEM((1, N), f32),
        ],
    )(msgs, dstp)
    u3 = ut[:GF].reshape(G, FIN, N)
    embt = pl.pallas_call(
        _gcn_mm_kernel,
        grid=(G,),
        in_specs=[
            pl.BlockSpec((1, FIN, N), lambda g: (g, 0, 0)),
            pl.BlockSpec((D_MODEL, FIN), lambda g: (0, 0)),
            pl.BlockSpec((D_MODEL, 1), lambda g: (0, 0)),
        ],
        out_specs=pl.BlockSpec((1, D_MODEL, 1), lambda g: (g, 0, 0)),
        out_shape=jax.ShapeDtypeStruct((G, D_MODEL, 1), f32),
    )(u3, gwt, gbcol)
    emb = embt.reshape(G, D_MODEL)

    out128 = pl.pallas_call(
        _mamba_kernel,
        out_shape=jax.ShapeDtypeStruct((BZ, D_MODEL), f32),
    )(emb, ln_g, ln_b, in_proj, conv_w, conv_b, xp_dt, xp_b, xp_c, dt_proj_W,
      dt_bias, alogt, Dp, out_proj, rW1, rb1r, rw2r, rb2r, xW1, xb1r, xw2r,
      xb2r)
    return out128[:, :2]


# speedup vs baseline: 7.7949x; 1.1350x over previous
import functools

import jax
import jax.numpy as jnp
from jax import lax
from jax.experimental import pallas as pl
from jax.experimental.pallas import tpu as pltpu
from jax.experimental.pallas import tpu_sc as plsc

N = 2048
FIN = 3
G = 64
GF = G * FIN
GFP = 256
E = 4096
EA = E + N
D_MODEL = 128
D_STATE = 32
D_CONV = 4
D_INNER = 256
N_LAYERS = 3
DT_RANK = 8
BZ, SL = 4, 16
NC, NS = 2, 16
EPS = EA // (NC * NS)
ECH = 512
NECH = EA // ECH


def _prep_kernel(xrow_ref, dst_ref, m_ref, gcol_ref, bcol_ref, xs_ref):
    xr = xrow_ref[...]
    mm = m_ref[...]
    mean = jnp.dot(xr, mm, preferred_element_type=jnp.float32,
        precision=lax.Precision.HIGHEST)
    d = xr - mean
    var = jnp.dot(d * d, mm, preferred_element_type=jnp.float32,
        precision=lax.Precision.HIGHEST)
    xln = d * 1.0 / jnp.sqrt(var + 1e-5) * gcol_ref[...] + bcol_ref[...]
    dstv = dst_ref[...]
    chunks = []
    for k in range(N // 128):
        nid = lax.broadcasted_iota(jnp.int32, (128, 1), 0) + (k * 128)
        cnt = jnp.sum((dstv == nid).astype(jnp.float32), axis=1, keepdims=True)
        chunks.append(cnt)
    deg = jnp.concatenate(chunks, axis=0) + 1.0
    dinv = 1.0 / jnp.sqrt(deg)
    xs_ref[...] = jnp.concatenate(
        [xln * dinv, jnp.zeros((N, GFP - GF), jnp.float32)], axis=1)


def _sc_gather(xs, srcp):
    mesh = plsc.VectorSubcoreMesh(core_axis_name="c", subcore_axis_name="s")

    @functools.partial(
        pl.kernel,
        out_type=jax.ShapeDtypeStruct((EA, GFP), jnp.float32),
        mesh=mesh,
        scratch_types=[
            pltpu.VMEM((EPS,), jnp.int32),
            pltpu.VMEM((EPS, GFP), jnp.float32),
            pltpu.SemaphoreType.DMA,
        ],
    )
    def gat(xs_hbm, src_hbm, out_hbm, srcv, rows, sem):
        c = lax.axis_index("c")
        s = lax.axis_index("s")
        base = (c * NS + s) * EPS
        pltpu.sync_copy(src_hbm.at[pl.ds(base, EPS)], srcv)
        h1 = pltpu.async_copy(
            xs_hbm.at[srcv.at[pl.ds(0, 128)]], rows.at[pl.ds(0, 128)], sem)
        h2 = pltpu.async_copy(
            xs_hbm.at[srcv.at[pl.ds(128, EPS - 128)]],
            rows.at[pl.ds(128, EPS - 128)], sem)
        h1.wait()
        h2.wait()
        pltpu.sync_copy(rows, out_hbm.at[pl.ds(base, EPS)])

    return gat(xs, srcp)


def _agg_mm_kernel(msgs_ref, dstp_ref, ut_ref, acc_ref, deg_ref):
    j = pl.program_id(0)

    @pl.when(j == 0)
    def _init():
        acc_ref[...] = jnp.zeros_like(acc_ref)
        deg_ref[...] = jnp.zeros_like(deg_ref)

    dstc = dstp_ref[0]
    onehot = (dstc == lax.broadcasted_iota(jnp.int32, (ECH, N), 1)
              ).astype(jnp.float32)
    acc_ref[...] += lax.dot_general(
        msgs_ref[:, :GF], onehot, (((0,), (0,)), ((), ())),
        preferred_element_type=jnp.float32,
        precision=lax.Precision.HIGHEST)
    deg_ref[...] += jnp.sum(onehot, axis=0, keepdims=True)

    @pl.when(j == NECH - 1)
    def _fin():
        ut_ref[...] = acc_ref[...] * 1.0 / jnp.sqrt(deg_ref[...])


GB = 8


def _gcn_mm_kernel(u3_ref, gwt_ref, gb_ref, emb_ref):
    for g in range(GB):
        xg = u3_ref[g]
        ht = jnp.dot(gwt_ref[...], xg, preferred_element_type=jnp.float32,
            precision=lax.Precision.HIGHEST)
        ht = jnp.maximum(ht + gb_ref[...], 0.0)
        emb_ref[g] = jnp.sum(ht, axis=1, keepdims=True) * (1.0 / N)


def _silu(v):
    return v * jax.nn.sigmoid(v)


def _softplus(v):
    return jnp.maximum(v, 0.0) + jnp.log(1.0 + jnp.exp(-jnp.abs(v)))


def _mamba_kernel(emb_ref, lng_ref, lnb_ref, inproj_ref, convw_ref, convb_ref,
                  xpdt_ref, xpb_ref, xpc_ref, dtw_ref, dtb_ref, alogt_ref,
                  dp_ref, outproj_ref, rw1_ref, rb1_ref, rw2_ref, rb2_ref,
                  xw1_ref, xb1_ref, xw2_ref, xb2_ref, out_ref):
    t = emb_ref[...]
    for l in range(N_LAYERS):
        mean = jnp.mean(t, axis=1, keepdims=True)
        d = t - mean
        var = jnp.mean(d * d, axis=1, keepdims=True)
        tln = d * 1.0 / jnp.sqrt(var + 1e-5) * lng_ref[l][None, :] + lnb_ref[l][None, :]
        xz = jnp.dot(tln, inproj_ref[l], preferred_element_type=jnp.float32,
        precision=lax.Precision.HIGHEST)
        xi, z = xz[:, :D_INNER], xz[:, D_INNER:]
        xi3 = xi.reshape(BZ, SL, D_INNER)
        conv = xi3 * convw_ref[l, :, D_CONV - 1][None, None, :]
        for k in range(D_CONV - 1):
            sh = D_CONV - 1 - k
            shifted = jnp.concatenate(
                [jnp.zeros((BZ, sh, D_INNER), jnp.float32), xi3[:, :SL - sh, :]],
                axis=1)
            conv = conv + shifted * convw_ref[l, :, k][None, None, :]
        xi2 = _silu(conv.reshape(G, D_INNER) + convb_ref[l][None, :])
        dtr = jnp.dot(xi2, xpdt_ref[l], preferred_element_type=jnp.float32,
        precision=lax.Precision.HIGHEST)
        dt = _softplus(jnp.dot(dtr, dtw_ref[l], preferred_element_type=jnp.float32,
        precision=lax.Precision.HIGHEST)
                       + dtb_ref[l][None, :])
        bc = jnp.dot(xi2, xpb_ref[l], preferred_element_type=jnp.float32,
        precision=lax.Precision.HIGHEST)
        cc = jnp.dot(xi2, xpc_ref[l], preferred_element_type=jnp.float32,
        precision=lax.Precision.HIGHEST)
        at = -jnp.exp(alogt_ref[l])
        dt3 = dt.reshape(BZ, SL, D_INNER)
        u3 = xi2.reshape(BZ, SL, D_INNER)
        bc3 = bc.reshape(BZ, SL, D_STATE)
        cc3 = cc.reshape(BZ, SL, D_STATE)
        h = jnp.zeros((BZ, D_STATE, D_INNER), jnp.float32)
        ys = []
        for s_ in range(SL):
            dts = dt3[:, s_, :][:, None, :]
            da = jnp.exp(dts * at[None])
            us = u3[:, s_, :][:, None, :]
            bs = bc3[:, s_, :][:, :, None]
            h = da * h + dts * bs * us
            cs = cc3[:, s_, :][:, :, None]
            ys.append(jnp.sum(h * cs, axis=1))
        y = jnp.stack(ys, axis=1).reshape(G, D_INNER)
        y = y + xi2 * dp_ref[l][None, :]
        y = y * _silu(z)
        t = t + jnp.dot(y, outproj_ref[l], preferred_element_type=jnp.float32,
        precision=lax.Precision.HIGHEST)
    final = t.reshape(BZ, SL, D_MODEL)[:, SL - 1, :]
    f1 = jnp.maximum(jnp.dot(final, rw1_ref[...],
                             preferred_element_type=jnp.float32,
        precision=lax.Precision.HIGHEST)
                     + rb1_ref[...], 0.0)
    r = jnp.sum(f1 * rw2_ref[...], axis=1, keepdims=True) + rb2_ref[...]
    f2 = jnp.maximum(jnp.dot(final, xw1_ref[...],
                             preferred_element_type=jnp.float32,
        precision=lax.Precision.HIGHEST)
                     + xb1_ref[...], 0.0)
    xp = jnp.sum(f2 * xw2_ref[...], axis=1, keepdims=True) + xb2_ref[...]
    out_ref[...] = jnp.concatenate(
        [r, xp, jnp.zeros((BZ, D_MODEL - 2), jnp.float32)], axis=1)


def kernel(x, edge_index, nrm_g, nrm_b, gW1, gb1, ln_g, ln_b, in_proj, conv_w,
           conv_b, x_proj, dt_proj_W, dt_bias, A_log, Dp, rW1, rb1, rW2, rb2,
           xW1, xb1, xW2, xb2, out_proj):
    f32 = jnp.float32
    xrow = x.reshape(G, N, FIN).transpose(1, 0, 2).reshape(N, GF)
    src = edge_index[0]
    dst = edge_index[1]
    dst_row = dst.reshape(1, E)
    nid = jnp.arange(N, dtype=jnp.int32)
    srcp = jnp.concatenate([src, nid])
    dstp = jnp.concatenate([dst, nid]).reshape(NECH, ECH, 1)
    eye = jnp.eye(G, dtype=f32)
    mmat = jnp.repeat(jnp.repeat(eye, FIN, axis=0), FIN, axis=1) * (1.0 / FIN)
    gcol = jnp.tile(nrm_g, G).reshape(1, GF)
    bcol = jnp.tile(nrm_b, G).reshape(1, GF)
    gwt = gW1.T
    gbcol = gb1.reshape(D_MODEL, 1)
    alogt = A_log.transpose(0, 2, 1)
    xp_dt = x_proj[:, :, :DT_RANK]
    xp_b = x_proj[:, :, DT_RANK:DT_RANK + D_STATE]
    xp_c = x_proj[:, :, DT_RANK + D_STATE:]
    rb1r = rb1.reshape(1, D_MODEL // 2)
    rw2r = rW2.reshape(1, D_MODEL // 2)
    rb2r = rb2.reshape(1, 1)
    xb1r = xb1.reshape(1, D_MODEL // 2)
    xw2r = xW2.reshape(1, D_MODEL // 2)
    xb2r = xb2.reshape(1, 1)

    xs = pl.pallas_call(
        _prep_kernel,
        out_shape=jax.ShapeDtypeStruct((N, GFP), f32),
    )(xrow, dst_row, mmat, gcol, bcol)

    msgs = _sc_gather(xs, srcp)

    ut = pl.pallas_call(
        _agg_mm_kernel,
        grid=(NECH,),
        in_specs=[
            pl.BlockSpec((ECH, GFP), lambda j: (j, 0)),
            pl.BlockSpec((1, ECH, 1), lambda j: (j, 0, 0)),
        ],
        out_specs=pl.BlockSpec((GF, N), lambda j: (0, 0)),
        out_shape=jax.ShapeDtypeStruct((GF, N), f32),
        scratch_shapes=[
            pltpu.VMEM((GF, N), f32),
            pltpu.VMEM((1, N), f32),
        ],
    )(msgs, dstp)
    u3 = ut.reshape(G, FIN, N)
    embt = pl.pallas_call(
        _gcn_mm_kernel,
        grid=(G // GB,),
        in_specs=[
            pl.BlockSpec((GB, FIN, N), lambda g: (g, 0, 0)),
            pl.BlockSpec((D_MODEL, FIN), lambda g: (0, 0)),
            pl.BlockSpec((D_MODEL, 1), lambda g: (0, 0)),
        ],
        out_specs=pl.BlockSpec((GB, D_MODEL, 1), lambda g: (g, 0, 0)),
        out_shape=jax.ShapeDtypeStruct((G, D_MODEL, 1), f32),
    )(u3, gwt, gbcol)
    emb = embt.reshape(G, D_MODEL)

    out128 = pl.pallas_call(
        _mamba_kernel,
        out_shape=jax.ShapeDtypeStruct((BZ, D_MODEL), f32),
    )(emb, ln_g, ln_b, in_proj, conv_w, conv_b, xp_dt, xp_b, xp_c, dt_proj_W,
      dt_bias, alogt, Dp, out_proj, rW1, rb1r, rw2r, rb2r, xW1, xb1r, xw2r,
      xb2r)
    return out128[:, :2]


# speedup vs baseline: 10.6305x; 1.3638x over previous
import functools

import jax
import jax.numpy as jnp
from jax import lax
from jax.experimental import pallas as pl
from jax.experimental.pallas import tpu as pltpu
from jax.experimental.pallas import tpu_sc as plsc

N = 2048
FIN = 3
G = 64
GF = G * FIN
GFP = 256
E = 4096
EA = E + N
D_MODEL = 128
D_STATE = 32
D_CONV = 4
D_INNER = 256
N_LAYERS = 3
DT_RANK = 8
BZ, SL = 4, 16
NC, NS = 2, 16
EPS = EA // (NC * NS)
ECH = 512
NECH = EA // ECH


def _prep_kernel(xrow_ref, dst_ref, m_ref, gcol_ref, bcol_ref, xs_ref):
    xr = xrow_ref[...]
    mm = m_ref[...]
    mean = jnp.dot(xr, mm, preferred_element_type=jnp.float32,
        precision=lax.Precision.HIGHEST)
    d = xr - mean
    var = jnp.dot(d * d, mm, preferred_element_type=jnp.float32,
        precision=lax.Precision.HIGHEST)
    xln = d * 1.0 / jnp.sqrt(var + 1e-5) * gcol_ref[...] + bcol_ref[...]
    dstv = dst_ref[...]
    chunks = []
    for k in range(N // 128):
        nid = lax.broadcasted_iota(jnp.int32, (128, 1), 0) + (k * 128)
        cnt = jnp.sum((dstv == nid).astype(jnp.float32), axis=1, keepdims=True)
        chunks.append(cnt)
    deg = jnp.concatenate(chunks, axis=0) + 1.0
    dinv = 1.0 / jnp.sqrt(deg)
    xs_ref[...] = jnp.concatenate(
        [xln * dinv, jnp.zeros((N, GFP - GF), jnp.float32)], axis=1)


def _sc_gather(xs, srcp):
    mesh = plsc.VectorSubcoreMesh(core_axis_name="c", subcore_axis_name="s")

    @functools.partial(
        pl.kernel,
        out_type=jax.ShapeDtypeStruct((EA, GFP), jnp.float32),
        mesh=mesh,
        scratch_types=[
            pltpu.VMEM((EPS,), jnp.int32),
            pltpu.VMEM((EPS, GFP), jnp.float32),
            pltpu.SemaphoreType.DMA,
        ],
    )
    def gat(xs_hbm, src_hbm, out_hbm, srcv, rows, sem):
        c = lax.axis_index("c")
        s = lax.axis_index("s")
        base = (c * NS + s) * EPS
        pltpu.sync_copy(src_hbm.at[pl.ds(base, EPS)], srcv)
        h1 = pltpu.async_copy(
            xs_hbm.at[srcv.at[pl.ds(0, 128)]], rows.at[pl.ds(0, 128)], sem)
        h2 = pltpu.async_copy(
            xs_hbm.at[srcv.at[pl.ds(128, EPS - 128)]],
            rows.at[pl.ds(128, EPS - 128)], sem)
        h1.wait()
        h2.wait()
        pltpu.sync_copy(rows, out_hbm.at[pl.ds(base, EPS)])

    return gat(xs, srcp)


def _agg_mm_kernel(msgs_ref, dstp_ref, ut_ref, acc_ref, deg_ref):
    j = pl.program_id(0)

    @pl.when(j == 0)
    def _init():
        acc_ref[...] = jnp.zeros_like(acc_ref)
        deg_ref[...] = jnp.zeros_like(deg_ref)

    dstc = dstp_ref[0]
    onehotf = (dstc == lax.broadcasted_iota(jnp.int32, (ECH, N), 1)
               ).astype(jnp.float32)
    onehot = onehotf.astype(jnp.bfloat16)
    m = msgs_ref[:, :GF]
    mh = m.astype(jnp.bfloat16)
    r1 = m - mh.astype(jnp.float32)
    ml = r1.astype(jnp.bfloat16)
    mll = (r1 - ml.astype(jnp.float32)).astype(jnp.bfloat16)
    dn = (((0,), (0,)), ((), ()))
    acc_ref[...] += (
        lax.dot_general(mh, onehot, dn, preferred_element_type=jnp.float32)
        + lax.dot_general(ml, onehot, dn, preferred_element_type=jnp.float32)
        + lax.dot_general(mll, onehot, dn, preferred_element_type=jnp.float32))
    deg_ref[...] += jnp.sum(onehotf, axis=0, keepdims=True)

    @pl.when(j == NECH - 1)
    def _fin():
        ut_ref[...] = acc_ref[...] * 1.0 / jnp.sqrt(deg_ref[...])


GB = 8


def _gcn_mm_kernel(u3_ref, gwt_ref, gb_ref, emb_ref):
    for g in range(GB):
        ht = gb_ref[...]
        for f in range(FIN):
            ht = ht + gwt_ref[:, f:f + 1] * u3_ref[g, f:f + 1, :]
        ht = jnp.maximum(ht, 0.0)
        emb_ref[g] = jnp.sum(ht, axis=1, keepdims=True) * (1.0 / N)


def _silu(v):
    return v * jax.nn.sigmoid(v)


def _softplus(v):
    return jnp.maximum(v, 0.0) + jnp.log(1.0 + jnp.exp(-jnp.abs(v)))


def _mamba_kernel(emb_ref, lng_ref, lnb_ref, inproj_ref, convw_ref, convb_ref,
                  xpdt_ref, xpb_ref, xpc_ref, dtw_ref, dtb_ref, alogt_ref,
                  dp_ref, outproj_ref, rw1_ref, rb1_ref, rw2_ref, rb2_ref,
                  xw1_ref, xb1_ref, xw2_ref, xb2_ref, out_ref):
    t = emb_ref[...]
    for l in range(N_LAYERS):
        mean = jnp.mean(t, axis=1, keepdims=True)
        d = t - mean
        var = jnp.mean(d * d, axis=1, keepdims=True)
        tln = d * 1.0 / jnp.sqrt(var + 1e-5) * lng_ref[l][None, :] + lnb_ref[l][None, :]
        xz = jnp.dot(tln, inproj_ref[l], preferred_element_type=jnp.float32,
        precision=lax.Precision.HIGHEST)
        xi, z = xz[:, :D_INNER], xz[:, D_INNER:]
        xi3 = xi.reshape(BZ, SL, D_INNER)
        conv = xi3 * convw_ref[l, :, D_CONV - 1][None, None, :]
        for k in range(D_CONV - 1):
            sh = D_CONV - 1 - k
            shifted = jnp.concatenate(
                [jnp.zeros((BZ, sh, D_INNER), jnp.float32), xi3[:, :SL - sh, :]],
                axis=1)
            conv = conv + shifted * convw_ref[l, :, k][None, None, :]
        xi2 = _silu(conv.reshape(G, D_INNER) + convb_ref[l][None, :])
        dtr = jnp.dot(xi2, xpdt_ref[l], preferred_element_type=jnp.float32,
        precision=lax.Precision.HIGHEST)
        dt = _softplus(jnp.dot(dtr, dtw_ref[l], preferred_element_type=jnp.float32,
        precision=lax.Precision.HIGHEST)
                       + dtb_ref[l][None, :])
        bc = jnp.dot(xi2, xpb_ref[l], preferred_element_type=jnp.float32,
        precision=lax.Precision.HIGHEST)
        cc = jnp.dot(xi2, xpc_ref[l], preferred_element_type=jnp.float32,
        precision=lax.Precision.HIGHEST)
        at = -jnp.exp(alogt_ref[l])
        dt3 = dt.reshape(BZ, SL, D_INNER)
        u3 = xi2.reshape(BZ, SL, D_INNER)
        bc3 = bc.reshape(BZ, SL, D_STATE)
        cc3 = cc.reshape(BZ, SL, D_STATE)
        h = jnp.zeros((BZ, D_STATE, D_INNER), jnp.float32)
        ys = []
        for s_ in range(SL):
            dts = dt3[:, s_, :][:, None, :]
            da = jnp.exp(dts * at[None])
            us = u3[:, s_, :][:, None, :]
            bs = bc3[:, s_, :][:, :, None]
            h = da * h + dts * bs * us
            cs = cc3[:, s_, :][:, :, None]
            ys.append(jnp.sum(h * cs, axis=1))
        y = jnp.stack(ys, axis=1).reshape(G, D_INNER)
        y = y + xi2 * dp_ref[l][None, :]
        y = y * _silu(z)
        t = t + jnp.dot(y, outproj_ref[l], preferred_element_type=jnp.float32,
        precision=lax.Precision.HIGHEST)
    final = t.reshape(BZ, SL, D_MODEL)[:, SL - 1, :]
    f1 = jnp.maximum(jnp.dot(final, rw1_ref[...],
                             preferred_element_type=jnp.float32,
        precision=lax.Precision.HIGHEST)
                     + rb1_ref[...], 0.0)
    r = jnp.sum(f1 * rw2_ref[...], axis=1, keepdims=True) + rb2_ref[...]
    f2 = jnp.maximum(jnp.dot(final, xw1_ref[...],
                             preferred_element_type=jnp.float32,
        precision=lax.Precision.HIGHEST)
                     + xb1_ref[...], 0.0)
    xp = jnp.sum(f2 * xw2_ref[...], axis=1, keepdims=True) + xb2_ref[...]
    out_ref[...] = jnp.concatenate(
        [r, xp, jnp.zeros((BZ, D_MODEL - 2), jnp.float32)], axis=1)


def kernel(x, edge_index, nrm_g, nrm_b, gW1, gb1, ln_g, ln_b, in_proj, conv_w,
           conv_b, x_proj, dt_proj_W, dt_bias, A_log, Dp, rW1, rb1, rW2, rb2,
           xW1, xb1, xW2, xb2, out_proj):
    f32 = jnp.float32
    xrow = x.reshape(G, N, FIN).transpose(1, 0, 2).reshape(N, GF)
    src = edge_index[0]
    dst = edge_index[1]
    dst_row = dst.reshape(1, E)
    nid = jnp.arange(N, dtype=jnp.int32)
    srcp = jnp.concatenate([src, nid])
    dstp = jnp.concatenate([dst, nid]).reshape(NECH, ECH, 1)
    eye = jnp.eye(G, dtype=f32)
    mmat = jnp.repeat(jnp.repeat(eye, FIN, axis=0), FIN, axis=1) * (1.0 / FIN)
    gcol = jnp.tile(nrm_g, G).reshape(1, GF)
    bcol = jnp.tile(nrm_b, G).reshape(1, GF)
    gwt = gW1.T
    gbcol = gb1.reshape(D_MODEL, 1)
    alogt = A_log.transpose(0, 2, 1)
    xp_dt = x_proj[:, :, :DT_RANK]
    xp_b = x_proj[:, :, DT_RANK:DT_RANK + D_STATE]
    xp_c = x_proj[:, :, DT_RANK + D_STATE:]
    rb1r = rb1.reshape(1, D_MODEL // 2)
    rw2r = rW2.reshape(1, D_MODEL // 2)
    rb2r = rb2.reshape(1, 1)
    xb1r = xb1.reshape(1, D_MODEL // 2)
    xw2r = xW2.reshape(1, D_MODEL // 2)
    xb2r = xb2.reshape(1, 1)

    xs = pl.pallas_call(
        _prep_kernel,
        out_shape=jax.ShapeDtypeStruct((N, GFP), f32),
    )(xrow, dst_row, mmat, gcol, bcol)

    msgs = _sc_gather(xs, srcp)

    ut = pl.pallas_call(
        _agg_mm_kernel,
        grid=(NECH,),
        in_specs=[
            pl.BlockSpec((ECH, GFP), lambda j: (j, 0)),
            pl.BlockSpec((1, ECH, 1), lambda j: (j, 0, 0)),
        ],
        out_specs=pl.BlockSpec((GF, N), lambda j: (0, 0)),
        out_shape=jax.ShapeDtypeStruct((GF, N), f32),
        scratch_shapes=[
            pltpu.VMEM((GF, N), f32),
            pltpu.VMEM((1, N), f32),
        ],
    )(msgs, dstp)
    u3 = ut.reshape(G, FIN, N)
    embt = pl.pallas_call(
        _gcn_mm_kernel,
        grid=(G // GB,),
        in_specs=[
            pl.BlockSpec((GB, FIN, N), lambda g: (g, 0, 0)),
            pl.BlockSpec((D_MODEL, FIN), lambda g: (0, 0)),
            pl.BlockSpec((D_MODEL, 1), lambda g: (0, 0)),
        ],
        out_specs=pl.BlockSpec((GB, D_MODEL, 1), lambda g: (g, 0, 0)),
        out_shape=jax.ShapeDtypeStruct((G, D_MODEL, 1), f32),
    )(u3, gwt, gbcol)
    emb = embt.reshape(G, D_MODEL)

    out128 = pl.pallas_call(
        _mamba_kernel,
        out_shape=jax.ShapeDtypeStruct((BZ, D_MODEL), f32),
    )(emb, ln_g, ln_b, in_proj, conv_w, conv_b, xp_dt, xp_b, xp_c, dt_proj_W,
      dt_bias, alogt, Dp, out_proj, rW1, rb1r, rw2r, rb2r, xW1, xb1r, xw2r,
      xb2r)
    return out128[:, :2]
